# baseline (speedup 1.0000x reference)
import functools

import jax
import jax.numpy as jnp
from jax import lax
from jax.experimental import pallas as pl
from jax.experimental.pallas import tpu as pltpu
from jax.experimental.pallas import tpu_sc as plsc

N = 10000
NP = 10240
D = 128
E = 320000
NC, NS = 2, 16
NW = NC * NS
CH = 128
EPT = 10240
EP = NW * EPT
NCHUNK = EPT // CH
CPS = 40
RPS = NP // NS
BM = 512
NB = NP // BM
HW = 16

_SC_MESH = dict(core_axis_name="c", subcore_axis_name="s")


def _hist_body(dst_hbm, zeros_hbm, deg_out, didx_v, hist_v, tmp_v, acc_v, deg_sh):
    cid = lax.axis_index("c")
    sid = lax.axis_index("s")
    wid = cid * NS + sid
    pltpu.sync_copy(zeros_hbm, hist_v)
    base = pl.multiple_of(wid * EPT, EPT)
    pltpu.sync_copy(dst_hbm.at[pl.ds(base, EPT)], didx_v)
    ones16 = jnp.ones((16,), jnp.float32)

    def step(j, c):
        iv = didx_v[pl.ds(j * 16, 16)]
        plsc.addupdate_scatter(hist_v, [iv], ones16)
        return c

    lax.fori_loop(0, EPT // 16, step, 0)
    pltpu.sync_copy(hist_v, deg_sh.at[sid])
    plsc.subcore_barrier()
    row0 = pl.multiple_of(sid * RPS, RPS)
    pltpu.sync_copy(deg_sh.at[0, pl.ds(row0, RPS)], acc_v)

    def tsum(t, c):
        pltpu.sync_copy(deg_sh.at[t, pl.ds(row0, RPS)], tmp_v)

        def vadd(v, c2):
            sl = pl.ds(v * 16, 16)
            acc_v[sl] = acc_v[sl] + tmp_v[sl]
            return c2

        lax.fori_loop(0, RPS // 16, vadd, 0)
        return c

    lax.fori_loop(1, NS, tsum, 0)
    pltpu.sync_copy(acc_v, deg_out.at[cid, pl.ds(row0, RPS)])


def _sc_hist(dst_p, zeros_c):
    k = pl.kernel(
        _hist_body,
        out_type=jax.ShapeDtypeStruct((NC, NP), jnp.float32),
        mesh=plsc.VectorSubcoreMesh(**_SC_MESH),
        compiler_params=pltpu.CompilerParams(needs_layout_passes=False),
        scratch_types=[
            pltpu.VMEM((EPT,), jnp.int32),
            pltpu.VMEM((NP,), jnp.float32),
            pltpu.VMEM((RPS,), jnp.float32),
            pltpu.VMEM((RPS,), jnp.float32),
            pltpu.VMEM_SHARED((NS, NP), jnp.float32),
        ],
    )
    return k(dst_p, zeros_c)


def _agg_body(g_hbm, src_hbm, dst_hbm, zeros_hbm, acc_out,
              sblk_v, dblk_v, rows_a, rows_b, acc_sh, sem_a, sem_b):
    cid = lax.axis_index("c")
    sid = lax.axis_index("s")
    wid = cid * NS + sid
    row0 = pl.multiple_of(sid * RPS, RPS)
    for j in range(RPS // CH):
        pltpu.sync_copy(zeros_hbm, acc_sh.at[pl.ds(row0 + j * CH, CH)])
    plsc.subcore_barrier()

    def seg(si, c0):
        pltpu.sync_copy(src_hbm.at[wid, pl.ds(si * ACPS, ACPS)], sblk_v)
        pltpu.sync_copy(dst_hbm.at[wid, pl.ds(si * ACPS, ACPS)], dblk_v)
        pltpu.async_copy(g_hbm.at[sblk_v.at[0]], rows_a, sem_a)

        def duo(t, c):
            pltpu.async_copy(g_hbm.at[sblk_v.at[2 * t + 1]], rows_b, sem_b)
            pltpu.make_async_copy(g_hbm.at[sblk_v.at[2 * t]], rows_a, sem_a).wait()
            pltpu.sync_copy(rows_a, acc_sh.at[dblk_v.at[2 * t]], add=True)

            @pl.when(t < ACPS // 2 - 1)
            def _():
                pltpu.async_copy(g_hbm.at[sblk_v.at[2 * t + 2]], rows_a, sem_a)

            pltpu.make_async_copy(g_hbm.at[sblk_v.at[2 * t + 1]], rows_b, sem_b).wait()
            pltpu.sync_copy(rows_b, acc_sh.at[dblk_v.at[2 * t + 1]], add=True)
            return c

        lax.fori_loop(0, ACPS // 2, duo, 0)
        return c0

    lax.fori_loop(0, ANCH // ACPS, seg, 0)
    plsc.subcore_barrier()
    pltpu.sync_copy(acc_sh.at[pl.ds(row0, RPS)], acc_out.at[cid, pl.ds(row0, RPS)])


ACH = 64
ANCH = EPT // ACH
ACPS = ANCH // 2


def _sc_agg(g, src_p, dst_p, zeros2d):
    src3 = jnp.reshape(src_p, (NW, ANCH, ACH))
    dst3 = jnp.reshape(dst_p, (NW, ANCH, ACH))
    k = pl.kernel(
        _agg_body,
        out_type=jax.ShapeDtypeStruct((NC, NP, D), jnp.float32),
        mesh=plsc.VectorSubcoreMesh(**_SC_MESH),
        scratch_types=[
            pltpu.VMEM((ACPS, ACH), jnp.int32),
            pltpu.VMEM((ACPS, ACH), jnp.int32),
            pltpu.VMEM((ACH, D), jnp.float32),
            pltpu.VMEM((ACH, D), jnp.float32),
            pltpu.VMEM_SHARED((NP, D), jnp.float32),
            pltpu.SemaphoreType.DMA,
            pltpu.SemaphoreType.DMA,
        ],
    )
    return k(g, src3, dst3, zeros2d)


def _mm_body(x_ref, w_ref, dp_ref, g_ref):
    h = jnp.dot(x_ref[...], w_ref[...], preferred_element_type=jnp.float32)
    s = dp_ref[0] + dp_ref[1] + 1.0
    g_ref[...] = h * lax.rsqrt(s)


def _tc_matmul(x, W, deg_parts):
    return pl.pallas_call(
        _mm_body,
        grid=(NB,),
        in_specs=[
            pl.BlockSpec((BM, D), lambda i: (i, 0)),
            pl.BlockSpec((D, D), lambda i: (0, 0)),
            pl.BlockSpec((NC, BM, 1), lambda i: (0, i, 0)),
        ],
        out_specs=pl.BlockSpec((BM, D), lambda i: (i, 0)),
        out_shape=jax.ShapeDtypeStruct((N, D), jnp.float32),
    )(x, W, deg_parts)


def _epa_body(ap_ref, g_ref, dp_ref, b_ref, u_ref, st_ref):
    s = dp_ref[0] + dp_ref[1] + 1.0
    dis = lax.rsqrt(s)
    acc = ap_ref[0] + ap_ref[1]
    u = dis * (acc + g_ref[...]) + b_ref[...]
    u_ref[...] = u
    i = pl.program_id(0)
    rid = i * BM + lax.broadcasted_iota(jnp.int32, (BM, 1), 0)
    um = jnp.where(rid < N, u, 0.0)
    s1 = jnp.sum(um, axis=0)[None, None, :]
    s2 = jnp.sum(um * um, axis=0)[None, None, :]
    st_ref[...] = jnp.concatenate([s1, s2], axis=1)


def _tc_epi_a(acc_parts, g, deg_parts, b2):
    return pl.pallas_call(
        _epa_body,
        grid=(NB,),
        in_specs=[
            pl.BlockSpec((NC, BM, D), lambda i: (0, i, 0)),
            pl.BlockSpec((BM, D), lambda i: (i, 0)),
            pl.BlockSpec((NC, BM, 1), lambda i: (0, i, 0)),
            pl.BlockSpec((1, D), lambda i: (0, 0)),
        ],
        out_specs=[
            pl.BlockSpec((BM, D), lambda i: (i, 0)),
            pl.BlockSpec((1, 2, D), lambda i: (i, 0, 0)),
        ],
        out_shape=[
            jax.ShapeDtypeStruct((N, D), jnp.float32),
            jax.ShapeDtypeStruct((NB, 2, D), jnp.float32),
        ],
    )(acc_parts, g, deg_parts, b2)


def _epb_body(u_ref, st_ref, gam_ref, bet_ref, o_ref):
    st = st_ref[...]
    inv_n = 1.0 / N
    mean = jnp.sum(st[:, 0, :], axis=0) * inv_n
    ex2 = jnp.sum(st[:, 1, :], axis=0) * inv_n
    var = ex2 - mean * mean
    scale = lax.rsqrt(var + 1e-5)[None, :] * gam_ref[...]
    o = (u_ref[...] - mean[None, :]) * scale + bet_ref[...]
    o_ref[...] = jnp.maximum(o, 0.0)


def _tc_epi_b(u, stats, gamma2, beta2):
    return pl.pallas_call(
        _epb_body,
        grid=(NB,),
        in_specs=[
            pl.BlockSpec((BM, D), lambda i: (i, 0)),
            pl.BlockSpec((NB, 2, D), lambda i: (0, 0, 0)),
            pl.BlockSpec((1, D), lambda i: (0, 0)),
            pl.BlockSpec((1, D), lambda i: (0, 0)),
        ],
        out_specs=pl.BlockSpec((BM, D), lambda i: (i, 0)),
        out_shape=jax.ShapeDtypeStruct((N, D), jnp.float32),
    )(u, stats, gamma2, beta2)


def kernel(x, edge_index, W, b, gamma, beta):
    src = edge_index[0].astype(jnp.int32)
    dst = edge_index[1].astype(jnp.int32)
    pad = EP - E
    src_p = jnp.concatenate([src, jnp.zeros((pad,), jnp.int32)])
    dst_p = jnp.concatenate([dst, jnp.full((pad,), N, jnp.int32)])

    zeros_np = jnp.zeros((NP,), jnp.float32)
    zeros2d = jnp.zeros((CH, D), jnp.float32)

    deg_parts = _sc_hist(dst_p, zeros_np)
    deg_parts = jnp.reshape(deg_parts, (NC, NP, 1))
    g = _tc_matmul(x, W, deg_parts)
    acc_parts = _sc_agg(g, src_p, dst_p, zeros2d)
    b2 = jnp.reshape(b, (1, D))
    u, stats = _tc_epi_a(acc_parts, g, deg_parts, b2)
    gamma2 = jnp.reshape(gamma, (1, D))
    beta2 = jnp.reshape(beta, (1, D))
    return _tc_epi_b(u, stats, gamma2, beta2)


# speedup vs baseline: 1.0573x; 1.0573x over previous
import functools

import jax
import jax.numpy as jnp
from jax import lax
from jax.experimental import pallas as pl
from jax.experimental.pallas import tpu as pltpu
from jax.experimental.pallas import tpu_sc as plsc

N = 10000
NP = 10240
D = 128
E = 320000
NC, NS = 2, 16
NW = NC * NS
CH = 128
EPT = 10240
EP = NW * EPT
NCHUNK = EPT // CH
CPS = 40
RPS = NP // NS
BM = 512
NB = NP // BM
HW = 16

_SC_MESH = dict(core_axis_name="c", subcore_axis_name="s")


def _hist_body(dst_hbm, zeros_hbm, deg_out, didx_v, hist_v, tmp_v, acc_v, deg_sh):
    cid = lax.axis_index("c")
    sid = lax.axis_index("s")
    wid = cid * NS + sid
    pltpu.sync_copy(zeros_hbm, hist_v)
    base = pl.multiple_of(wid * EPT, EPT)
    pltpu.sync_copy(dst_hbm.at[pl.ds(base, EPT)], didx_v)
    ones16 = jnp.ones((16,), jnp.float32)

    def step(j, c):
        iv = didx_v[pl.ds(j * 16, 16)]
        plsc.addupdate_scatter(hist_v, [iv], ones16)
        return c

    lax.fori_loop(0, EPT // 16, step, 0)
    pltpu.sync_copy(hist_v, deg_sh.at[sid])
    plsc.subcore_barrier()
    row0 = pl.multiple_of(sid * RPS, RPS)
    pltpu.sync_copy(deg_sh.at[0, pl.ds(row0, RPS)], acc_v)

    def tsum(t, c):
        pltpu.sync_copy(deg_sh.at[t, pl.ds(row0, RPS)], tmp_v)

        def vadd(v, c2):
            sl = pl.ds(v * 16, 16)
            acc_v[sl] = acc_v[sl] + tmp_v[sl]
            return c2

        lax.fori_loop(0, RPS // 16, vadd, 0)
        return c

    lax.fori_loop(1, NS, tsum, 0)
    pltpu.sync_copy(acc_v, deg_out.at[cid, pl.ds(row0, RPS)])


def _sc_hist(dst_p, zeros_c):
    k = pl.kernel(
        _hist_body,
        out_type=jax.ShapeDtypeStruct((NC, NP), jnp.float32),
        mesh=plsc.VectorSubcoreMesh(**_SC_MESH),
        compiler_params=pltpu.CompilerParams(needs_layout_passes=False),
        scratch_types=[
            pltpu.VMEM((EPT,), jnp.int32),
            pltpu.VMEM((NP,), jnp.float32),
            pltpu.VMEM((RPS,), jnp.float32),
            pltpu.VMEM((RPS,), jnp.float32),
            pltpu.VMEM_SHARED((NS, NP), jnp.float32),
        ],
    )
    return k(dst_p, zeros_c)


def _agg_body(g_hbm, src_hbm, dst_hbm, zeros_hbm, acc_out,
              sblk_v, dblk_v, rows_a, rows_b, acc_sh, sem_a, sem_b):
    cid = lax.axis_index("c")
    sid = lax.axis_index("s")
    wid = cid * NS + sid
    row0 = pl.multiple_of(sid * RPS, RPS)
    for j in range(RPS // CH):
        pltpu.sync_copy(zeros_hbm, acc_sh.at[pl.ds(row0 + j * CH, CH)])
    plsc.subcore_barrier()

    def seg(si, c0):
        pltpu.sync_copy(src_hbm.at[wid, pl.ds(si * ACPS, ACPS)], sblk_v)
        pltpu.sync_copy(dst_hbm.at[wid, pl.ds(si * ACPS, ACPS)], dblk_v)
        pltpu.async_copy(g_hbm.at[sblk_v.at[0]], rows_a, sem_a)

        def duo(t, c):
            pltpu.async_copy(g_hbm.at[sblk_v.at[2 * t + 1]], rows_b, sem_b)
            pltpu.make_async_copy(g_hbm.at[sblk_v.at[2 * t]], rows_a, sem_a).wait()
            pltpu.sync_copy(rows_a, acc_sh.at[dblk_v.at[2 * t]], add=True)

            @pl.when(t < ACPS // 2 - 1)
            def _():
                pltpu.async_copy(g_hbm.at[sblk_v.at[2 * t + 2]], rows_a, sem_a)

            pltpu.make_async_copy(g_hbm.at[sblk_v.at[2 * t + 1]], rows_b, sem_b).wait()
            pltpu.sync_copy(rows_b, acc_sh.at[dblk_v.at[2 * t + 1]], add=True)
            return c

        lax.fori_loop(0, ACPS // 2, duo, 0)
        return c0

    lax.fori_loop(0, ANCH // ACPS, seg, 0)
    plsc.subcore_barrier()
    pltpu.sync_copy(acc_sh.at[pl.ds(row0, RPS)], acc_out.at[cid, pl.ds(row0, RPS)])


ACH = 128
ANCH = EPT // ACH
ACPS = ANCH // 2


def _sc_agg(g, src_p, dst_p, zeros2d):
    src3 = jnp.reshape(src_p, (NW, ANCH, ACH))
    dst3 = jnp.reshape(dst_p, (NW, ANCH, ACH))
    k = pl.kernel(
        _agg_body,
        out_type=jax.ShapeDtypeStruct((NC, NP, D), jnp.float32),
        mesh=plsc.VectorSubcoreMesh(**_SC_MESH),
        scratch_types=[
            pltpu.VMEM((ACPS, ACH), jnp.int32),
            pltpu.VMEM((ACPS, ACH), jnp.int32),
            pltpu.VMEM((ACH, D), jnp.float32),
            pltpu.VMEM((ACH, D), jnp.float32),
            pltpu.VMEM_SHARED((NP, D), jnp.float32),
            pltpu.SemaphoreType.DMA,
            pltpu.SemaphoreType.DMA,
        ],
    )
    return k(g, src3, dst3, zeros2d)


def _mm_body(x_ref, w_ref, dp_ref, g_ref):
    h = jnp.dot(x_ref[...], w_ref[...], preferred_element_type=jnp.float32)
    s = dp_ref[0] + dp_ref[1] + 1.0
    g_ref[...] = h * lax.rsqrt(s)


def _tc_matmul(x, W, deg_parts):
    return pl.pallas_call(
        _mm_body,
        grid=(NB,),
        in_specs=[
            pl.BlockSpec((BM, D), lambda i: (i, 0)),
            pl.BlockSpec((D, D), lambda i: (0, 0)),
            pl.BlockSpec((NC, BM, 1), lambda i: (0, i, 0)),
        ],
        out_specs=pl.BlockSpec((BM, D), lambda i: (i, 0)),
        out_shape=jax.ShapeDtypeStruct((N, D), jnp.float32),
    )(x, W, deg_parts)


def _ep_body(ap_ref, g_ref, dp_ref, b_ref, gam_ref, bet_ref, o_ref, u_sc, st_sc):
    i = pl.program_id(0)
    blk = lax.rem(i, NB)
    row = pl.multiple_of(blk * BM, BM)

    @pl.when(i == 0)
    def _():
        st_sc[...] = jnp.zeros((8, D), jnp.float32)

    @pl.when(i < NB)
    def _():
        s = dp_ref[0] + dp_ref[1] + 1.0
        dis = lax.rsqrt(s)
        acc = ap_ref[0] + ap_ref[1]
        u = dis * (acc + g_ref[...]) + b_ref[...]
        u_sc[pl.ds(row, BM), :] = u
        rid = blk * BM + lax.broadcasted_iota(jnp.int32, (BM, 1), 0)
        um = jnp.where(rid < N, u, 0.0)
        st_sc[0:1, :] = st_sc[0:1, :] + jnp.sum(um, axis=0)[None, :]
        st_sc[1:2, :] = st_sc[1:2, :] + jnp.sum(um * um, axis=0)[None, :]

    @pl.when(i >= NB)
    def _():
        inv_n = 1.0 / N
        mean = st_sc[0:1, :] * inv_n
        ex2 = st_sc[1:2, :] * inv_n
        var = ex2 - mean * mean
        scale = lax.rsqrt(var + 1e-5) * gam_ref[...]
        u = u_sc[pl.ds(row, BM), :]
        o_ref[...] = jnp.maximum((u - mean) * scale + bet_ref[...], 0.0)


def _tc_epilogue(acc_parts, g, deg_parts, b2, gamma2, beta2):
    return pl.pallas_call(
        _ep_body,
        grid=(2 * NB,),
        in_specs=[
            pl.BlockSpec((NC, BM, D), lambda i: (0, i % NB, 0)),
            pl.BlockSpec((BM, D), lambda i: (i % NB, 0)),
            pl.BlockSpec((NC, BM, 1), lambda i: (0, i % NB, 0)),
            pl.BlockSpec((1, D), lambda i: (0, 0)),
            pl.BlockSpec((1, D), lambda i: (0, 0)),
            pl.BlockSpec((1, D), lambda i: (0, 0)),
        ],
        out_specs=pl.BlockSpec((BM, D), lambda i: (i % NB, 0)),
        out_shape=jax.ShapeDtypeStruct((N, D), jnp.float32),
        scratch_shapes=[
            pltpu.VMEM((NP, D), jnp.float32),
            pltpu.VMEM((8, D), jnp.float32),
        ],
    )(acc_parts, g, deg_parts, b2, gamma2, beta2)


def kernel(x, edge_index, W, b, gamma, beta):
    src = edge_index[0].astype(jnp.int32)
    dst = edge_index[1].astype(jnp.int32)
    pad = EP - E
    src_p = jnp.concatenate([src, jnp.zeros((pad,), jnp.int32)])
    dst_p = jnp.concatenate([dst, jnp.full((pad,), N, jnp.int32)])

    zeros_np = jnp.zeros((NP,), jnp.float32)
    zeros2d = jnp.zeros((CH, D), jnp.float32)

    deg_parts = _sc_hist(dst_p, zeros_np)
    deg_parts = jnp.reshape(deg_parts, (NC, NP, 1))
    g = _tc_matmul(x, W, deg_parts)
    acc_parts = _sc_agg(g, src_p, dst_p, zeros2d)
    b2 = jnp.reshape(b, (1, D))
    gamma2 = jnp.reshape(gamma, (1, D))
    beta2 = jnp.reshape(beta, (1, D))
    return _tc_epilogue(acc_parts, g, deg_parts, b2, gamma2, beta2)
